# Initial kernel scaffold; baseline (speedup 1.0000x reference)
import jax
import jax.numpy as jnp
from jax.experimental import pallas as pl


def kernel(points, feats, neighbor_indices, W, b):
    raise NotImplementedError("write your pallas kernel here")


# speedup vs baseline: 1.7985x; 1.7985x over previous
import functools

import jax
import jax.numpy as jnp
from jax import lax
from jax.experimental import pallas as pl
from jax.experimental.pallas import tpu as pltpu
from jax.experimental.pallas import tpu_sc as plsc

N = 10000
K = 16
C = 256
LANES = 16
PTS_W = 4

NW = 32
NP = 10240
PW = NP // NW
CH = 8
NCHUNK = PW // CH
CV = C // LANES


def _mm_body(x_ref, wt_ref, b_ref, o_ref):
    o_ref[...] = (
        jnp.dot(x_ref[...], wt_ref[...], preferred_element_type=jnp.float32)
        + b_ref[...]
    )


def _linear(feats_pad, Wt, b):
    grid = NP // 1024
    return pl.pallas_call(
        _mm_body,
        grid=(grid,),
        in_specs=[
            pl.BlockSpec((1024, C), lambda i: (i, 0)),
            pl.BlockSpec((C, C), lambda i: (0, 0)),
            pl.BlockSpec((1, C), lambda i: (0, 0)),
        ],
        out_specs=pl.BlockSpec((1024, C), lambda i: (i, 0)),
        out_shape=jax.ShapeDtypeStruct((NP, C), jnp.float32),
    )(feats_pad, Wt, b)


def _sc_attend_body(f_hbm, ni_hbm, pts_hbm, out_hbm,
                    idx_v, nbrf_v, pts_v, q_v, out_v, acc_v, red_v, sem_f):
    wid = lax.axis_index("s") * 2 + lax.axis_index("c")
    base = wid * PW
    lane_ids = lax.iota(jnp.int32, LANES)
    zeros_i = jnp.zeros((LANES,), jnp.int32)

    pltpu.sync_copy(pts_hbm, pts_v)

    def lane_sum(v):
        for s in (8, 4, 2, 1):
            red_v[0, :] = v
            v = v + plsc.load_gather(red_v, [zeros_i, lane_ids ^ s])
        return v

    def lane_max(v):
        for s in (8, 4, 2, 1):
            red_v[0, :] = v
            v = jnp.maximum(v, plsc.load_gather(red_v, [zeros_i, lane_ids ^ s]))
        return v

    def chunk_body(g, _):
        row0 = base + g * CH
        pltpu.sync_copy(ni_hbm.at[pl.ds(row0 * K, CH * K)], idx_v)
        cp_f = pltpu.async_copy(f_hbm.at[idx_v], nbrf_v, sem_f)
        pltpu.sync_copy(f_hbm.at[pl.ds(row0, CH)], q_v)
        cp_f.wait()

        def point_body(p, _):
            qs = [q_v[p, pl.ds(c * LANES, LANES)] for c in range(CV)]
            for k in range(K):
                acc = qs[0] * nbrf_v[p * K + k, pl.ds(0, LANES)]
                for c in range(1, CV):
                    acc = acc + qs[c] * nbrf_v[p * K + k, pl.ds(c * LANES, LANES)]
                acc_v[k, :] = acc
            dots = plsc.load_gather(acc_v, [lane_ids, zeros_i])
            for l in range(1, LANES):
                dots = dots + plsc.load_gather(acc_v, [lane_ids, zeros_i + l])
            s = dots * (1.0 / 16.0)
            e = jnp.exp(s - lane_max(s))
            w = e / lane_sum(e)
            nidx = idx_v[pl.ds(p * K, LANES)] * PTS_W
            ox = lane_sum(w * plsc.load_gather(pts_v, [nidx]))
            oy = lane_sum(w * plsc.load_gather(pts_v, [nidx + 1]))
            oz = lane_sum(w * plsc.load_gather(pts_v, [nidx + 2]))
            zero = jnp.zeros((LANES,), jnp.float32)
            ov = (jnp.where(lane_ids == 0, ox, zero)
                  + jnp.where(lane_ids == 1, oy, zero)
                  + jnp.where(lane_ids == 2, oz, zero))
            out_v[p, :] = ov
            return 0

        lax.fori_loop(0, CH, point_body, 0)
        pltpu.sync_copy(out_v, out_hbm.at[pl.ds(row0, CH)])
        return 0

    lax.fori_loop(0, NCHUNK, chunk_body, 0)


_sc_attend = functools.partial(
    pl.kernel,
    mesh=plsc.VectorSubcoreMesh(core_axis_name="c", subcore_axis_name="s"),
    compiler_params=pltpu.CompilerParams(needs_layout_passes=False),
    out_type=jax.ShapeDtypeStruct((NP, LANES), jnp.float32),
    scratch_types=[
        pltpu.VMEM((CH * K,), jnp.int32),
        pltpu.VMEM((CH * K, C), jnp.float32),
        pltpu.VMEM((N * PTS_W,), jnp.float32),
        pltpu.VMEM((CH, C), jnp.float32),
        pltpu.VMEM((CH, LANES), jnp.float32),
        pltpu.VMEM((K, LANES), jnp.float32),
        pltpu.VMEM((1, LANES), jnp.float32),
        pltpu.SemaphoreType.DMA,
    ],
)(_sc_attend_body)


@jax.jit
def kernel(points, feats, neighbor_indices, W, b):
    ni = neighbor_indices.astype(jnp.int32)
    own = jnp.broadcast_to(jnp.arange(N, dtype=jnp.int32)[:, None], (N, K))
    ni = jnp.where(ni < N, ni, own)
    ni_flat = jnp.pad(ni.reshape(-1), (0, (NP - N) * K))

    feats_pad = jnp.pad(feats, ((0, NP - N), (0, 0)))
    f = _linear(feats_pad, W.T, b[None, :])

    pts_pad = jnp.pad(points, ((0, 0), (0, PTS_W - 3))).reshape(-1)
    out = _sc_attend(f, ni_flat, pts_pad)
    return out[:N, :3]


# speedup vs baseline: 2.6281x; 1.4612x over previous
import functools

import jax
import jax.numpy as jnp
from jax import lax
from jax.experimental import pallas as pl
from jax.experimental.pallas import tpu as pltpu
from jax.experimental.pallas import tpu_sc as plsc

N = 10000
K = 16
C = 256
LANES = 16
PTS_W = 4

NW = 32
NP = 10240
PW = NP // NW
CH = 8
NCHUNK = PW // CH
LAST = NCHUNK - 1
CV = C // LANES


def _mm_body(x_ref, wt_ref, b_ref, o_ref):
    o_ref[...] = (
        jnp.dot(x_ref[...], wt_ref[...], preferred_element_type=jnp.float32)
        + b_ref[...]
    )


def _linear(feats_pad, Wt, b):
    grid = NP // 1024
    return pl.pallas_call(
        _mm_body,
        grid=(grid,),
        in_specs=[
            pl.BlockSpec((1024, C), lambda i: (i, 0)),
            pl.BlockSpec((C, C), lambda i: (0, 0)),
            pl.BlockSpec((1, C), lambda i: (0, 0)),
        ],
        out_specs=pl.BlockSpec((1024, C), lambda i: (i, 0)),
        out_shape=jax.ShapeDtypeStruct((NP, C), jnp.float32),
    )(feats_pad, Wt, b)


def _tree_sum(vs):
    while len(vs) > 1:
        nxt = [vs[i] + vs[i + 1] for i in range(0, len(vs) - 1, 2)]
        if len(vs) % 2:
            nxt.append(vs[-1])
        vs = nxt
    return vs[0]


def _sc_attend_body(f_hbm, ni_hbm, pts_hbm, out_hbm,
                    idx_a, idx_b, nbrf_a, nbrf_b, q_a, q_b, out_a, out_b,
                    pts_v, accf_v, nidx_s,
                    sem_ni_a, sem_ni_b, sem_g_a, sem_g_b,
                    sem_q_a, sem_q_b, sem_o_a, sem_o_b):
    wid = lax.axis_index("s") * 2 + lax.axis_index("c")
    base = wid * PW
    lane_ids = lax.iota(jnp.int32, LANES)
    row_base = lane_ids * LANES

    bufs = [
        (idx_a, nbrf_a, q_a, out_a, sem_ni_a, sem_g_a, sem_q_a, sem_o_a),
        (idx_b, nbrf_b, q_b, out_b, sem_ni_b, sem_g_b, sem_q_b, sem_o_b),
    ]

    def ni_copy(c, idxr, sem):
        return pltpu.make_async_copy(
            ni_hbm.at[pl.ds((base + c * CH) * K, CH * K)], idxr, sem)

    def g_copy(idxr, nbr, sem):
        return pltpu.make_async_copy(f_hbm.at[idxr], nbr, sem)

    def q_copy(c, qr, sem):
        return pltpu.make_async_copy(
            f_hbm.at[pl.ds(base + c * CH, CH)], qr, sem)

    def o_copy(c, outr, sem):
        return pltpu.make_async_copy(
            outr, out_hbm.at[pl.ds(base + c * CH, CH)], sem)

    ni_copy(0, idx_a, sem_ni_a).start()
    ni_copy(1, idx_b, sem_ni_b).start()
    pltpu.sync_copy(pts_hbm, pts_v)
    ni_copy(0, idx_a, sem_ni_a).wait()
    g_copy(idx_a, nbrf_a, sem_g_a).start()
    q_copy(0, q_a, sem_q_a).start()

    def compute_chunk(nidx_s, nbrX, qX, outX):
        def point_body(p, _):
            qs = [qX[p, pl.ds(c * LANES, LANES)] for c in range(CV)]
            for k in range(K):
                rk = [nbrX[p * K + k, pl.ds(c * LANES, LANES)]
                      for c in range(CV)]
                acc0 = qs[0] * rk[0]
                acc1 = qs[1] * rk[1]
                for c in range(2, CV, 2):
                    acc0 = acc0 + qs[c] * rk[c]
                for c in range(3, CV, 2):
                    acc1 = acc1 + qs[c] * rk[c]
                accf_v[pl.ds(k * LANES, LANES)] = acc0 + acc1
            dots = _tree_sum(
                [plsc.load_gather(accf_v, [row_base + l]) for l in range(LANES)])
            e = jnp.exp(dots * (1.0 / 16.0))
            nidx = nidx_s[pl.ds(p * K, LANES)] * PTS_W
            px = plsc.load_gather(pts_v, [nidx])
            py = plsc.load_gather(pts_v, [nidx + 1])
            pz = plsc.load_gather(pts_v, [nidx + 2])
            accf_v[pl.ds(0, LANES)] = e * px
            accf_v[pl.ds(LANES, LANES)] = e * py
            accf_v[pl.ds(2 * LANES, LANES)] = e * pz
            accf_v[pl.ds(3 * LANES, LANES)] = e
            t = _tree_sum(
                [plsc.load_gather(accf_v, [row_base + l]) for l in range(LANES)])
            outX[p, :] = t / t[3]
            return 0

        lax.fori_loop(0, CH, point_body, 0)

    def step_body(s, _):
        for b in range(2):
            (idxX, nbrX, qX, outX, sem_niX, sem_gX, sem_qX, sem_oX) = bufs[b]
            (idxY, nbrY, qY, outY, sem_niY, sem_gY, sem_qY, sem_oY) = bufs[1 - b]
            g = 2 * s + b
            g1 = jnp.minimum(g + 1, LAST)
            g2 = jnp.minimum(g + 2, LAST)
            ni_copy(g1, idxY, sem_niY).wait()
            g_copy(idxX, nbrX, sem_gX).wait()
            q_copy(g, qX, sem_qX).wait()
            for j in range(CH):
                nidx_s[pl.ds(j * LANES, LANES)] = idxX[pl.ds(j * LANES, LANES)]
            ni_copy(g2, idxX, sem_niX).start()
            g_copy(idxY, nbrY, sem_gY).start()
            q_copy(g1, qY, sem_qY).start()

            @pl.when(g >= 2)
            def _():
                o_copy(g - 2, outX, sem_oX).wait()

            compute_chunk(nidx_s, nbrX, qX, outX)
            o_copy(g, outX, sem_oX).start()
        return 0

    lax.fori_loop(0, NCHUNK // 2, step_body, 0)

    ni_copy(LAST, idx_b, sem_ni_b).wait()
    g_copy(idx_a, nbrf_a, sem_g_a).wait()
    q_copy(LAST, q_a, sem_q_a).wait()
    o_copy(LAST - 1, out_a, sem_o_a).wait()
    o_copy(LAST, out_b, sem_o_b).wait()


_sc_attend = functools.partial(
    pl.kernel,
    mesh=plsc.VectorSubcoreMesh(core_axis_name="c", subcore_axis_name="s"),
    compiler_params=pltpu.CompilerParams(needs_layout_passes=False),
    out_type=jax.ShapeDtypeStruct((NP, LANES), jnp.float32),
    scratch_types=[
        pltpu.VMEM((CH * K,), jnp.int32),
        pltpu.VMEM((CH * K,), jnp.int32),
        pltpu.VMEM((CH * K, C), jnp.float32),
        pltpu.VMEM((CH * K, C), jnp.float32),
        pltpu.VMEM((CH, C), jnp.float32),
        pltpu.VMEM((CH, C), jnp.float32),
        pltpu.VMEM((CH, LANES), jnp.float32),
        pltpu.VMEM((CH, LANES), jnp.float32),
        pltpu.VMEM((N * PTS_W,), jnp.float32),
        pltpu.VMEM((K * LANES,), jnp.float32),
        pltpu.VMEM((CH * K,), jnp.int32),
        pltpu.SemaphoreType.DMA,
        pltpu.SemaphoreType.DMA,
        pltpu.SemaphoreType.DMA,
        pltpu.SemaphoreType.DMA,
        pltpu.SemaphoreType.DMA,
        pltpu.SemaphoreType.DMA,
        pltpu.SemaphoreType.DMA,
        pltpu.SemaphoreType.DMA,
    ],
)(_sc_attend_body)


@jax.jit
def kernel(points, feats, neighbor_indices, W, b):
    ni = neighbor_indices.astype(jnp.int32)
    own = jnp.broadcast_to(jnp.arange(N, dtype=jnp.int32)[:, None], (N, K))
    ni = jnp.where(ni < N, ni, own)
    ni_flat = jnp.pad(ni.reshape(-1), (0, (NP - N) * K))

    feats_pad = jnp.pad(feats, ((0, NP - N), (0, 0)))
    f = _linear(feats_pad, W.T, b[None, :])

    pts_pad = jnp.pad(points, ((0, 0), (0, PTS_W - 3))).reshape(-1)
    out = _sc_attend(f, ni_flat, pts_pad)
    return out[:N, :3]
